# Initial kernel scaffold; baseline (speedup 1.0000x reference)
import jax
import jax.numpy as jnp
from jax.experimental import pallas as pl


def kernel(idx, targets, table):
    raise NotImplementedError("write your pallas kernel here")


# speedup vs baseline: 1.4117x; 1.4117x over previous
import jax
import jax.numpy as jnp
from jax import lax
from jax.experimental import pallas as pl
from jax.experimental.pallas import tpu as pltpu
from jax.experimental.pallas import tpu_sc as plsc

VOCAB = 1000
N_TOK = 1024 * 50
NC, NS, L = 2, 16, 16
NW = NC * NS
PER_TILE = N_TOK // NW
CHUNK = 64
N_CHUNKS = PER_TILE // CHUNK
GROUPS = CHUNK // L


def _lse_body(table_ref, out_ref):
    t = table_ref[...]
    m = jnp.max(t, axis=1, keepdims=True)
    out_ref[...] = m[:, 0] + jnp.log(jnp.sum(jnp.exp(t - m), axis=1))


def _sc_body(idx_hbm, tgt_hbm, table_hbm, lse_hbm,
             out_hbm, loss_hbm,
             idx_v, tgt_v, lse_v, rows_v, accv, sums_v, lossv,
             shared, sem):
    cid = lax.axis_index("c")
    sid = lax.axis_index("s")
    wid = sid * NC + cid
    base = wid * PER_TILE

    pltpu.sync_copy(idx_hbm.at[pl.ds(base, PER_TILE)], idx_v)
    pltpu.sync_copy(tgt_hbm.at[pl.ds(base, PER_TILE)], tgt_v)
    pltpu.sync_copy(lse_hbm, lse_v)

    lanes = lax.iota(jnp.int32, L)

    def chunk(c, acc):
        off = c * CHUNK
        pltpu.async_copy(table_hbm.at[idx_v.at[pl.ds(off, CHUNK)]],
                         rows_v, sem).wait()
        for g in range(GROUPS):
            o = off + g * L
            rid = lanes + g * L
            tv = plsc.load_gather(rows_v, [rid, tgt_v[pl.ds(o, L)]])
            lv = plsc.load_gather(lse_v, [idx_v[pl.ds(o, L)]])
            acc = acc + lv - tv
        pltpu.sync_copy(rows_v, out_hbm.at[pl.ds(base + off, CHUNK)])
        return acc

    acc = lax.fori_loop(0, N_CHUNKS, chunk, jnp.zeros((L,), jnp.float32))
    accv[...] = acc
    pltpu.sync_copy(accv, shared.at[sid])
    plsc.subcore_barrier()

    @pl.when(sid == 0)
    def _():
        pltpu.sync_copy(shared, sums_v)
        tot = sums_v[0]
        for j in range(1, NS):
            tot = tot + sums_v[j]
        lossv[...] = tot * (1.0 / N_TOK)
        pltpu.sync_copy(lossv, loss_hbm.at[cid])


def kernel(idx, targets, table):
    lse = pl.pallas_call(
        _lse_body,
        out_shape=jax.ShapeDtypeStruct((VOCAB,), jnp.float32),
    )(table)

    mesh = plsc.VectorSubcoreMesh(core_axis_name="c", subcore_axis_name="s")
    sc = pl.kernel(
        _sc_body,
        out_type=[
            jax.ShapeDtypeStruct((N_TOK, VOCAB), jnp.float32),
            jax.ShapeDtypeStruct((NC, L), jnp.float32),
        ],
        mesh=mesh,
        compiler_params=pltpu.CompilerParams(use_tc_tiling_on_sc=False,
                                             needs_layout_passes=False),
        scratch_types=[
            pltpu.VMEM((PER_TILE,), jnp.int32),
            pltpu.VMEM((PER_TILE,), jnp.int32),
            pltpu.VMEM((VOCAB,), jnp.float32),
            pltpu.VMEM((CHUNK, VOCAB), jnp.float32),
            pltpu.VMEM((L,), jnp.float32),
            pltpu.VMEM((NS, L), jnp.float32),
            pltpu.VMEM((L,), jnp.float32),
            pltpu.VMEM_SHARED((NS, L), jnp.float32),
            pltpu.SemaphoreType.DMA,
        ],
    )
    flat_logits, loss_parts = sc(idx.reshape(N_TOK), targets.reshape(N_TOK),
                                 table, lse)
    logits = flat_logits.reshape(idx.shape[0], idx.shape[1], VOCAB)
    loss = jnp.sum(loss_parts)
    return (logits, loss)


# speedup vs baseline: 1.4228x; 1.0078x over previous
import jax
import jax.numpy as jnp
from jax import lax
from jax.experimental import pallas as pl
from jax.experimental.pallas import tpu as pltpu
from jax.experimental.pallas import tpu_sc as plsc

VOCAB = 1000
N_TOK = 1024 * 50
NC, NS, L = 2, 16, 16
NW = NC * NS
PER_TILE = N_TOK // NW
CHUNK = 32
N_CHUNKS = PER_TILE // CHUNK
GROUPS = CHUNK // L


def _lse_body(table_ref, out_ref):
    t = table_ref[...]
    m = jnp.max(t, axis=1, keepdims=True)
    out_ref[...] = m[:, 0] + jnp.log(jnp.sum(jnp.exp(t - m), axis=1))


def _sc_body(idx_hbm, tgt_hbm, table_hbm, lse_hbm,
             out_hbm, loss_hbm,
             idx_v, tgt_v, lse_v, rows_a, rows_b, accv, sums_v, lossv,
             shared, sem):
    cid = lax.axis_index("c")
    sid = lax.axis_index("s")
    wid = sid * NC + cid
    base = wid * PER_TILE

    pltpu.sync_copy(idx_hbm.at[pl.ds(base, PER_TILE)], idx_v)
    pltpu.sync_copy(tgt_hbm.at[pl.ds(base, PER_TILE)], tgt_v)
    pltpu.sync_copy(lse_hbm, lse_v)

    lanes = lax.iota(jnp.int32, L)
    rows = [rows_a, rows_b]

    def g_start(c, buf):
        off = c * CHUNK
        pltpu.async_copy(table_hbm.at[idx_v.at[pl.ds(off, CHUNK)]],
                         buf, sem)

    def g_wait(buf):
        pltpu.make_async_copy(table_hbm.at[idx_v.at[pl.ds(0, CHUNK)]],
                              buf, sem).wait()

    g_start(0, rows[0])

    def pair(p, acc):
        for b in range(2):
            c = p * 2 + b
            cur, nxt = rows[b], rows[1 - b]
            g_wait(cur)
            g_start(jnp.minimum(c + 1, N_CHUNKS - 1), nxt)
            off = c * CHUNK
            for g in range(GROUPS):
                o = off + g * L
                rid = lanes + g * L
                tv = plsc.load_gather(cur, [rid, tgt_v[pl.ds(o, L)]])
                lv = plsc.load_gather(lse_v, [idx_v[pl.ds(o, L)]])
                acc = acc + lv - tv
            pltpu.sync_copy(cur, out_hbm.at[pl.ds(base + off, CHUNK)])
        return acc

    acc = lax.fori_loop(0, N_CHUNKS // 2, pair, jnp.zeros((L,), jnp.float32))
    g_wait(rows[0])
    accv[...] = acc
    pltpu.sync_copy(accv, shared.at[sid])
    plsc.subcore_barrier()

    @pl.when(sid == 0)
    def _():
        pltpu.sync_copy(shared, sums_v)
        tot = sums_v[0]
        for j in range(1, NS):
            tot = tot + sums_v[j]
        lossv[...] = tot * (1.0 / N_TOK)
        pltpu.sync_copy(lossv, loss_hbm.at[cid])


def kernel(idx, targets, table):
    lse = pl.pallas_call(
        _lse_body,
        out_shape=jax.ShapeDtypeStruct((VOCAB,), jnp.float32),
    )(table)

    mesh = plsc.VectorSubcoreMesh(core_axis_name="c", subcore_axis_name="s")
    sc = pl.kernel(
        _sc_body,
        out_type=[
            jax.ShapeDtypeStruct((N_TOK, VOCAB), jnp.float32),
            jax.ShapeDtypeStruct((NC, L), jnp.float32),
        ],
        mesh=mesh,
        compiler_params=pltpu.CompilerParams(use_tc_tiling_on_sc=False,
                                             needs_layout_passes=False),
        scratch_types=[
            pltpu.VMEM((PER_TILE,), jnp.int32),
            pltpu.VMEM((PER_TILE,), jnp.int32),
            pltpu.VMEM((VOCAB,), jnp.float32),
            pltpu.VMEM((CHUNK, VOCAB), jnp.float32),
            pltpu.VMEM((CHUNK, VOCAB), jnp.float32),
            pltpu.VMEM((L,), jnp.float32),
            pltpu.VMEM((NS, L), jnp.float32),
            pltpu.VMEM((L,), jnp.float32),
            pltpu.VMEM_SHARED((NS, L), jnp.float32),
            pltpu.SemaphoreType.DMA,
        ],
    )
    flat_logits, loss_parts = sc(idx.reshape(N_TOK), targets.reshape(N_TOK),
                                 table, lse)
    logits = flat_logits.reshape(idx.shape[0], idx.shape[1], VOCAB)
    loss = jnp.sum(loss_parts)
    return (logits, loss)
